# baseline (speedup 1.0000x reference)
import jax
import jax.numpy as jnp
from jax.experimental import pallas as pl
from jax.experimental.pallas import tpu as pltpu
from jax.experimental.pallas import tpu_sc as plsc

NUM_E = 8192
DIM = 256
NT = 65536
TILE = 1024
EPS = 1e-12
GATHER_WIN = 128


def _prep_body(w_ref, wn_ref, wnb_ref, wsq_ref):
    w = w_ref[...]
    n = jnp.sqrt(jnp.sum(w * w, axis=1, keepdims=True))
    wn = w / jnp.maximum(n, EPS)
    wn_ref[...] = wn
    wnb_ref[...] = (wn * -2.0).astype(jnp.bfloat16)
    wsq_ref[...] = jnp.sum(wn * wn, axis=1, keepdims=True)


def _main_body(x_ref, wnb_ref, wsq_ref, idx_ref, loss_ref):
    i = pl.program_id(0)
    x = x_ref[...]
    n = jnp.sqrt(jnp.sum(x * x, axis=1, keepdims=True))
    xn = x / jnp.maximum(n, EPS)
    xsq = jnp.sum(xn * xn, axis=1, keepdims=True)
    s = jax.lax.dot_general(
        xn.astype(jnp.bfloat16), wnb_ref[...],
        (((1,), (1,)), ((), ())), preferred_element_type=jnp.float32)
    d = (xsq + wsq_ref[...]) + s
    half = NUM_E // 2
    da, db = d[:, :half], d[:, half:]
    ma = jnp.min(da, axis=1, keepdims=True)
    mb = jnp.min(db, axis=1, keepdims=True)
    ii = jax.lax.broadcasted_iota(jnp.int32, da.shape, 1)
    big = jnp.int32(NUM_E)
    ia = jnp.min(jnp.where(da <= ma, ii, big), axis=1, keepdims=True)
    ib = jnp.min(jnp.where(db <= mb, ii, big), axis=1, keepdims=True) + half
    take_a = ma.astype(jnp.bfloat16).astype(jnp.float32) <= mb
    idx_ref[...] = jnp.where(take_a, ia, ib)
    m = jnp.where(take_a, ma, mb)

    @pl.when(i == 0)
    def _():
        loss_ref[...] = jnp.zeros_like(loss_ref)

    loss_ref[...] += jnp.sum(m, axis=(0, 1), keepdims=True)


def _tc_stage(x, W):
    wn, wnb, wsq = pl.pallas_call(
        _prep_body,
        out_shape=(
            jax.ShapeDtypeStruct((NUM_E, DIM), jnp.float32),
            jax.ShapeDtypeStruct((NUM_E, DIM), jnp.bfloat16),
            jax.ShapeDtypeStruct((NUM_E, 1), jnp.float32),
        ),
    )(W)
    wsq_row = wsq.reshape(1, NUM_E)
    idx2, loss_sum = pl.pallas_call(
        _main_body,
        grid=(NT // TILE,),
        in_specs=[
            pl.BlockSpec((TILE, DIM), lambda i: (i, 0)),
            pl.BlockSpec((NUM_E, DIM), lambda i: (0, 0)),
            pl.BlockSpec((1, NUM_E), lambda i: (0, 0)),
        ],
        out_specs=[
            pl.BlockSpec((TILE, 1), lambda i: (i, 0)),
            pl.BlockSpec((1, 1), lambda i: (0, 0)),
        ],
        out_shape=(
            jax.ShapeDtypeStruct((NT, 1), jnp.int32),
            jax.ShapeDtypeStruct((1, 1), jnp.float32),
        ),
    )(x, wnb, wsq_row)
    return wn, idx2.reshape(NT), loss_sum[0, 0]


def _sc_gather(wn, idx_row):
    mesh = plsc.VectorSubcoreMesh(core_axis_name="c", subcore_axis_name="s")

    @pl.kernel(out_type=jax.ShapeDtypeStruct((NT, DIM), jnp.float32),
               mesh=mesh)
    def k(wn_hbm, i_hbm, o_hbm):
        def body(i_vmem, o_vmem):
            pltpu.sync_copy(wn_hbm.at[i_vmem.at[0]], o_vmem)

        pltpu.emit_pipeline(
            body,
            grid=(NT // GATHER_WIN,),
            in_specs=[pl.BlockSpec((1, GATHER_WIN), index_map=lambda i: (0, i))],
            out_specs=[pl.BlockSpec((GATHER_WIN, DIM), index_map=lambda i: (i, 0))],
            core_axis_name=("c", "s"),
            dimension_semantics=(pltpu.PARALLEL,),
        )(i_hbm, o_hbm)

    return k(wn, idx_row)


def kernel(x, W):
    wn, idx, loss_sum = _tc_stage(x, W)
    q = _sc_gather(wn, idx.reshape(1, NT))
    loss = loss_sum * (1.25 / (NT * DIM))
    return q, loss, idx


# speedup vs baseline: 1.0113x; 1.0113x over previous
import jax
import jax.numpy as jnp
from jax.experimental import pallas as pl
from jax.experimental.pallas import tpu as pltpu
from jax.experimental.pallas import tpu_sc as plsc

NUM_E = 8192
DIM = 256
NT = 65536
TILE = 1024
EPS = 1e-12
GATHER_WIN = 128


def _main_body(xn_ref, xsq_ref, wnb_ref, wsq_ref, idx_ref, loss_ref):
    i = pl.program_id(0)
    xsq = xsq_ref[...]
    s = jax.lax.dot_general(
        xn_ref[...].astype(jnp.bfloat16), wnb_ref[...],
        (((1,), (1,)), ((), ())), preferred_element_type=jnp.float32)
    d = (xsq + wsq_ref[...]) - 2.0 * s
    half = NUM_E // 2
    da, db = d[:, :half], d[:, half:]
    ma = jnp.min(da, axis=1, keepdims=True)
    mb = jnp.min(db, axis=1, keepdims=True)
    ii = jax.lax.broadcasted_iota(jnp.int32, da.shape, 1)
    big = jnp.int32(NUM_E)
    ia = jnp.min(jnp.where(da <= ma, ii, big), axis=1, keepdims=True)
    ib = jnp.min(jnp.where(db <= mb, ii, big), axis=1, keepdims=True) + half
    take_a = ma.astype(jnp.bfloat16).astype(jnp.float32) <= mb
    idx_ref[...] = jnp.where(take_a, ia, ib)
    m = jnp.where(take_a, ma, mb)

    @pl.when(i == 0)
    def _():
        loss_ref[...] = jnp.zeros_like(loss_ref)

    loss_ref[...] += jnp.sum(m, axis=(0, 1), keepdims=True)


def _tc_stage(xn, xsq, wnb, wsq_row):
    idx2, loss_sum = pl.pallas_call(
        _main_body,
        grid=(NT // TILE,),
        in_specs=[
            pl.BlockSpec((TILE, DIM), lambda i: (i, 0)),
            pl.BlockSpec((TILE, 1), lambda i: (i, 0)),
            pl.BlockSpec((NUM_E, DIM), lambda i: (0, 0)),
            pl.BlockSpec((1, NUM_E), lambda i: (0, 0)),
        ],
        out_specs=[
            pl.BlockSpec((TILE, 1), lambda i: (i, 0)),
            pl.BlockSpec((1, 1), lambda i: (0, 0)),
        ],
        out_shape=(
            jax.ShapeDtypeStruct((NT, 1), jnp.int32),
            jax.ShapeDtypeStruct((1, 1), jnp.float32),
        ),
    )(xn, xsq, wnb, wsq_row)
    return idx2.reshape(NT), loss_sum[0, 0]


def _sc_gather(wn, idx_row):
    mesh = plsc.VectorSubcoreMesh(core_axis_name="c", subcore_axis_name="s")

    @pl.kernel(out_type=jax.ShapeDtypeStruct((NT, DIM), jnp.float32),
               mesh=mesh)
    def k(wn_hbm, i_hbm, o_hbm):
        def body(i_vmem, o_vmem):
            pltpu.sync_copy(wn_hbm.at[i_vmem.at[0]], o_vmem)

        pltpu.emit_pipeline(
            body,
            grid=(NT // GATHER_WIN,),
            in_specs=[pl.BlockSpec((1, GATHER_WIN), index_map=lambda i: (0, i))],
            out_specs=[pl.BlockSpec((GATHER_WIN, DIM), index_map=lambda i: (i, 0))],
            core_axis_name=("c", "s"),
            dimension_semantics=(pltpu.PARALLEL,),
        )(i_hbm, o_hbm)

    return k(wn, idx_row)


def kernel(x, W):
    xn = x / jnp.maximum(jnp.sqrt(jnp.sum(x * x, axis=1, keepdims=True)), EPS)
    wn = W / jnp.maximum(jnp.sqrt(jnp.sum(W * W, axis=1, keepdims=True)), EPS)
    xsq = jnp.sum(xn * xn, axis=1, keepdims=True)
    wsq_row = jnp.sum(wn * wn, axis=1).reshape(1, NUM_E)
    wnb = wn.astype(jnp.bfloat16)
    idx, loss_sum = _tc_stage(xn, xsq, wnb, wsq_row)
    q = _sc_gather(wn, idx.reshape(1, NT))
    loss = loss_sum * (1.25 / (NT * DIM))
    return q, loss, idx


# speedup vs baseline: 1.1681x; 1.1551x over previous
import jax
import jax.numpy as jnp
from jax.experimental import pallas as pl
from jax.experimental.pallas import tpu as pltpu
from jax.experimental.pallas import tpu_sc as plsc

NUM_E = 8192
DIM = 256
NT = 65536
TILE = 1024
EPS = 1e-12
GATHER_WIN = 128


def _main_body(x_ref, wnb_ref, wsq_ref, idx_ref, loss_ref):
    i = pl.program_id(0)
    x = x_ref[...]
    n = jnp.sqrt(jnp.sum(x * x, axis=1, keepdims=True))
    xn = x / jnp.maximum(n, EPS)
    xsq = jnp.sum(xn * xn, axis=1, keepdims=True)
    s = jax.lax.dot_general(
        xn.astype(jnp.bfloat16), wnb_ref[...],
        (((1,), (1,)), ((), ())), preferred_element_type=jnp.float32)
    d = (xsq + wsq_ref[...]) - 2.0 * s
    half = NUM_E // 2
    da, db = d[:, :half], d[:, half:]
    ma = jnp.min(da, axis=1, keepdims=True)
    mb = jnp.min(db, axis=1, keepdims=True)
    ii = jax.lax.broadcasted_iota(jnp.int32, da.shape, 1)
    big = jnp.int32(NUM_E)
    ia = jnp.min(jnp.where(da <= ma, ii, big), axis=1, keepdims=True)
    ib = jnp.min(jnp.where(db <= mb, ii, big), axis=1, keepdims=True) + half
    take_a = ma.astype(jnp.bfloat16).astype(jnp.float32) <= mb
    idx_ref[...] = jnp.where(take_a, ia, ib)
    m = jnp.where(take_a, ma, mb)

    @pl.when(i == 0)
    def _():
        loss_ref[...] = jnp.zeros_like(loss_ref)

    loss_ref[...] += jnp.sum(m, axis=(0, 1), keepdims=True)


def _tc_stage(x, wnb, wsq_row):
    idx2, loss_sum = pl.pallas_call(
        _main_body,
        grid=(NT // TILE,),
        in_specs=[
            pl.BlockSpec((TILE, DIM), lambda i: (i, 0)),
            pl.BlockSpec((NUM_E, DIM), lambda i: (0, 0)),
            pl.BlockSpec((1, NUM_E), lambda i: (0, 0)),
        ],
        out_specs=[
            pl.BlockSpec((TILE, 1), lambda i: (i, 0)),
            pl.BlockSpec((1, 1), lambda i: (0, 0)),
        ],
        out_shape=(
            jax.ShapeDtypeStruct((NT, 1), jnp.int32),
            jax.ShapeDtypeStruct((1, 1), jnp.float32),
        ),
    )(x, wnb, wsq_row)
    return idx2.reshape(NT), loss_sum[0, 0]


def _sc_gather(wn, idx_row):
    mesh = plsc.VectorSubcoreMesh(core_axis_name="c", subcore_axis_name="s")

    @pl.kernel(out_type=jax.ShapeDtypeStruct((NT, DIM), jnp.float32),
               mesh=mesh)
    def k(wn_hbm, i_hbm, o_hbm):
        def body(i_vmem, o_vmem):
            pltpu.sync_copy(wn_hbm.at[i_vmem.at[0]], o_vmem)

        pltpu.emit_pipeline(
            body,
            grid=(NT // GATHER_WIN,),
            in_specs=[pl.BlockSpec((1, GATHER_WIN), index_map=lambda i: (0, i))],
            out_specs=[pl.BlockSpec((GATHER_WIN, DIM), index_map=lambda i: (i, 0))],
            core_axis_name=("c", "s"),
            dimension_semantics=(pltpu.PARALLEL,),
        )(i_hbm, o_hbm)

    return k(wn, idx_row)


def kernel(x, W):
    wn = W / jnp.maximum(jnp.sqrt(jnp.sum(W * W, axis=1, keepdims=True)), EPS)
    wsq_row = jnp.sum(wn * wn, axis=1).reshape(1, NUM_E)
    wnb = wn.astype(jnp.bfloat16)
    idx, loss_sum = _tc_stage(x, wnb, wsq_row)
    q = _sc_gather(wn, idx.reshape(1, NT))
    loss = loss_sum * (1.25 / (NT * DIM))
    return q, loss, idx
